# baseline (speedup 1.0000x reference)
import functools

import jax
import jax.numpy as jnp
import numpy as np
from jax import lax
from jax.experimental import pallas as pl
from jax.experimental.pallas import tpu as pltpu
from jax.experimental.pallas import tpu_sc as plsc

_PATCH = 32
_GRID = 10
_H2 = 16
_W2 = 16
_E = _PATCH * _PATCH * 3
_WHALF = 256
_WQ = 128


def _hash_rows():
    i = np.arange(_H2)
    j = np.arange(_W2)
    hi = np.floor(i.astype(np.float32) * _GRID / _H2).astype(np.int32)
    hj = np.floor(j.astype(np.float32) * _GRID / _W2).astype(np.int32)
    return (hi[:, None] * _GRID + hj[None, :]).reshape(-1)


def _sc_body(x_hbm, tab_hbm, idx_hbm, out_hbm,
             in0, in1, out0, out1, out2, shared, idx8_v,
             sem_i0, sem_i1, sem_o0, sem_o1, sem_o2,
             sem_p0, sem_p1, sem_p2, sem_t):
    cidx = lax.axis_index("c")
    sidx = lax.axis_index("s")
    wid = sidx * 2 + cidx
    ph = wid // 2
    half = wid % 2
    p0 = ph * _W2 + half * 8
    row0 = ph * _PATCH
    w0 = half * _WHALF
    nb = x_hbm.shape[0]

    def din(b, q, buf, sem):
        return pltpu.make_async_copy(
            x_hbm.at[b, :, pl.ds(row0, _PATCH), pl.ds(w0 + q * _WQ, _WQ)],
            buf, sem)

    def dout(b, buf, sem):
        return pltpu.make_async_copy(buf, out_hbm.at[b, pl.ds(p0, 8)], sem)

    def prefill(buf, sem):
        return pltpu.make_async_copy(shared.at[sidx], buf, sem)

    din(0, 0, in0, sem_i0).start()
    din(0, 1, in1, sem_i1).start()

    pltpu.sync_copy(idx_hbm.at[pl.ds(p0, 8)], idx8_v)
    pltpu.async_copy(tab_hbm.at[idx8_v], out0, sem_t).wait()
    pltpu.sync_copy(out0, shared.at[sidx])

    prefill(out0, sem_p0).start()
    prefill(out1, sem_p1).start()
    prefill(out2, sem_p2).start()

    iota3 = lax.iota(jnp.int32, 16) * 3
    rows = [jnp.full((16,), pw, jnp.int32) for pw in range(8)]

    def compute(in_v, out_v, q):
        for c in range(3):
            @plsc.parallel_loop(0, _PATCH, unroll=2)
            def _(kh, c=c):
                col0 = iota3 + (kh * 96 + c)
                col1 = col0 + 48
                for s in range(8):
                    row = rows[q * 4 + s // 2]
                    col = col0 if s % 2 == 0 else col1
                    v = in_v[c, kh, pl.ds(s * 16, 16)]
                    plsc.addupdate_scatter(out_v, [row, col], v)

    def phase(b, out_v, sem_o, sem_p):
        din(0, 0, in0, sem_i0).wait()
        prefill(out_v, sem_p).wait()
        compute(in0, out_v, 0)

        @pl.when(b + 1 < nb)
        def _():
            din(b + 1, 0, in0, sem_i0).start()

        din(0, 1, in1, sem_i1).wait()
        compute(in1, out_v, 1)
        dout(b, out_v, sem_o).start()

        @pl.when(b + 1 < nb)
        def _():
            din(b + 1, 1, in1, sem_i1).start()

    outs = [(out0, sem_o0, sem_p0), (out1, sem_o1, sem_p1),
            (out2, sem_o2, sem_p2)]

    phase(0, *outs[0])
    phase(1, *outs[1])

    def hexad(i, carry):
        for k in range(6):
            b = 6 * i + 2 + k
            o_v, o_sem, p_sem = outs[(2 + k) % 3]
            d_v, d_sem, dp_sem = outs[k % 3]
            dout(0, d_v, d_sem).wait()
            prefill(d_v, dp_sem).start()
            phase(b, o_v, o_sem, p_sem)
        return carry

    lax.fori_loop(0, (nb - 2) // 6, hexad, 0)

    dout(0, out0, sem_o0).wait()
    dout(0, out1, sem_o1).wait()
    prefill(out2, sem_p2).wait()


def kernel(x, position_embeddings):
    b = x.shape[0]
    table = position_embeddings.reshape(_GRID * _GRID, _E)
    idx = jnp.asarray(_hash_rows())

    mesh = plsc.VectorSubcoreMesh(core_axis_name="c", subcore_axis_name="s")
    run = functools.partial(
        pl.kernel,
        out_type=jax.ShapeDtypeStruct((b, _H2 * _W2, _E), x.dtype),
        mesh=mesh,
        compiler_params=pltpu.CompilerParams(needs_layout_passes=False),
        scratch_types=[
            pltpu.VMEM((3, _PATCH, _WQ), jnp.float32),
            pltpu.VMEM((3, _PATCH, _WQ), jnp.float32),
            pltpu.VMEM((8, _E), jnp.float32),
            pltpu.VMEM((8, _E), jnp.float32),
            pltpu.VMEM((8, _E), jnp.float32),
            pltpu.VMEM_SHARED((16, 8, _E), jnp.float32),
            pltpu.VMEM((8,), jnp.int32),
            pltpu.SemaphoreType.DMA,
            pltpu.SemaphoreType.DMA,
            pltpu.SemaphoreType.DMA,
            pltpu.SemaphoreType.DMA,
            pltpu.SemaphoreType.DMA,
            pltpu.SemaphoreType.DMA,
            pltpu.SemaphoreType.DMA,
            pltpu.SemaphoreType.DMA,
            pltpu.SemaphoreType.DMA,
        ],
    )(_sc_body)
    return run(x, table, idx)
